# baseline (speedup 1.0000x reference)
import functools

import jax
import jax.numpy as jnp
from jax import lax
from jax.experimental import pallas as pl
from jax.experimental.pallas import tpu as pltpu
from jax.experimental.pallas import tpu_sc as plsc

DIM = 128
BATCH = 512
MARGIN = 0.0
L = 16
NC, NS = 2, 16
NW = NC * NS
RPW = BATCH // NW
NCHUNK = DIM // L


def _sc_body(cE, rE, nf1f, nf2f, nf3f, nf4f, djf, negf,
             o1, o2a, o2b, o3, o4, odj, oneg,
             idx3, crow, drow, erow, rrow, acc_a, acc_b, sem):
    cid = lax.axis_index("c")
    sid = lax.axis_index("s")
    wid = sid * NC + cid
    base = wid * RPW
    iota = lax.iota(jnp.int32, L)

    def fetch_idx(flat_ref, ncol):
        pltpu.sync_copy(flat_ref.at[pl.ds(base * ncol, RPW * ncol)],
                        idx3.at[pl.ds(0, RPW * ncol)])
        return [plsc.load_gather(idx3, [iota * ncol + j]) for j in range(ncol)]

    def gather2(i0, i1, table0=cE, table1=cE, buf0=None, buf1=None):
        cp0 = pltpu.make_async_copy(table0.at[i0], buf0, sem)
        cp1 = pltpu.make_async_copy(table1.at[i1], buf1, sem)
        cp0.start()
        cp1.start()
        cp0.wait()
        cp1.wait()

    def halfrow(ref, i, k, off):
        return ref[i, pl.ds(off + k * L, L)]

    def cc_loss(out_ref, sa, sb, use_r, r_sign):
        def row(i, _):
            def chunk(k, acc):
                c1 = halfrow(crow, i, k, 0)
                d1 = halfrow(drow, i, k, 0)
                co = jnp.abs(halfrow(crow, i, k, DIM))
                do = jnp.abs(halfrow(drow, i, k, DIM))
                cen = c1 - d1
                if use_r:
                    cen = cen + r_sign * rrow[i, pl.ds(k * L, L)]
                euc = jnp.abs(cen)
                t = jnp.maximum(sa * euc + sb[0] * co + sb[1] * do - MARGIN,
                                0.0)
                return acc + t * t
            acc = lax.fori_loop(0, NCHUNK, chunk, jnp.zeros((L,), jnp.float32),
                                unroll=True)
            acc_a[i, :] = acc
            return 0
        lax.fori_loop(0, RPW, row, 0)
        pltpu.sync_copy(acc_a, out_ref.at[pl.ds(base, RPW)])

    i0, i1 = fetch_idx(nf1f, 2)
    gather2(i0, i1, buf0=crow, buf1=drow)
    cc_loss(o1, 1.0, (1.0, -1.0), False, 0.0)

    i0, i1 = fetch_idx(djf, 2)
    gather2(i0, i1, buf0=crow, buf1=drow)
    cc_loss(odj, -1.0, (1.0, 1.0), False, 0.0)

    i0, i1, i2 = fetch_idx(nf3f, 3)
    cpr = pltpu.make_async_copy(rE.at[i1], rrow, sem)
    cpr.start()
    gather2(i0, i2, buf0=crow, buf1=drow)
    cpr.wait()
    cc_loss(o3, 1.0, (1.0, -1.0), True, 1.0)

    i0, i1, i2 = fetch_idx(negf, 3)
    cpr = pltpu.make_async_copy(rE.at[i1], rrow, sem)
    cpr.start()
    gather2(i0, i2, buf0=crow, buf1=drow)
    cpr.wait()
    cc_loss(oneg, 1.0, (-1.0, -1.0), True, 1.0)

    i0, i1, i2 = fetch_idx(nf4f, 3)
    cpr = pltpu.make_async_copy(rE.at[i0], rrow, sem)
    cpr.start()
    gather2(i1, i2, buf0=crow, buf1=drow)
    cpr.wait()
    cc_loss(o4, 1.0, (-1.0, -1.0), True, -1.0)

    i0, i1, i2 = fetch_idx(nf2f, 3)
    cp0 = pltpu.make_async_copy(cE.at[i0], crow, sem)
    cp1 = pltpu.make_async_copy(cE.at[i1], drow, sem)
    cp2 = pltpu.make_async_copy(cE.at[i2], erow, sem)
    cp0.start(); cp1.start(); cp2.start()
    cp0.wait(); cp1.wait(); cp2.wait()

    def nf2_row(i, _):
        def chunk(k, carry):
            aa, bb = carry
            c1 = halfrow(crow, i, k, 0)
            d1 = halfrow(drow, i, k, 0)
            e1 = halfrow(erow, i, k, 0)
            c2 = jnp.abs(halfrow(crow, i, k, DIM))
            d2 = jnp.abs(halfrow(drow, i, k, DIM))
            e2 = jnp.abs(halfrow(erow, i, k, DIM))
            start = jnp.maximum(c1 - c2, d1 - d2)
            end = jnp.minimum(c1 + c2, d1 + d2)
            diff = start - end
            new_r = jnp.abs(diff) * 0.5
            cen1 = (start + end) * 0.5
            u = jnp.maximum(jnp.abs(cen1 - e1) + new_r - e2 - MARGIN, 0.0)
            v = jnp.maximum(diff, 0.0)
            return aa + u * u, bb + v * v
        zero = jnp.zeros((L,), jnp.float32)
        aa, bb = lax.fori_loop(0, NCHUNK, chunk, (zero, zero), unroll=True)
        acc_a[i, :] = aa
        acc_b[i, :] = bb
        return 0
    lax.fori_loop(0, RPW, nf2_row, 0)
    pltpu.sync_copy(acc_a, o2a.at[pl.ds(base, RPW)])
    pltpu.sync_copy(acc_b, o2b.at[pl.ds(base, RPW)])


_part = jax.ShapeDtypeStruct((BATCH, L), jnp.float32)

_sc_kernel = pl.kernel(
    _sc_body,
    out_type=(_part,) * 7,
    mesh=plsc.VectorSubcoreMesh(core_axis_name="c", subcore_axis_name="s"),
    compiler_params=pltpu.CompilerParams(needs_layout_passes=False),
    scratch_types=[
        pltpu.VMEM((RPW * 3,), jnp.int32),
        pltpu.VMEM((RPW, 2 * DIM), jnp.float32),
        pltpu.VMEM((RPW, 2 * DIM), jnp.float32),
        pltpu.VMEM((RPW, 2 * DIM), jnp.float32),
        pltpu.VMEM((RPW, DIM), jnp.float32),
        pltpu.VMEM((RPW, L), jnp.float32),
        pltpu.VMEM((RPW, L), jnp.float32),
        pltpu.SemaphoreType.DMA,
    ],
)


def _finish_body(o1, o2a, o2b, o3, o4, odj, oneg, out):
    inv_b = 1.0 / BATCH
    loss1 = jnp.sum(o1[...]) * inv_b
    loss3 = jnp.sum(o3[...]) * inv_b
    loss4 = jnp.sum(o4[...]) * inv_b
    dj = jnp.sum(odj[...]) * inv_b
    a2 = jnp.sum(o2a[...], axis=1, keepdims=True)
    b2 = jnp.sum(o2b[...], axis=1, keepdims=True)
    mean_a = jnp.sum(jnp.sqrt(a2)) * inv_b
    mean_b = jnp.sum(jnp.sqrt(b2)) * inv_b
    loss2 = (jnp.sum(a2) + jnp.sum(b2)) * inv_b + 2.0 * mean_a * mean_b
    n2 = jnp.sum(oneg[...], axis=1, keepdims=True)
    dn = jnp.sqrt(n2)
    neg = jnp.sum((dn - 2.0) ** 2) * inv_b
    out[0, 0] = loss1 + loss2 + dj + loss3 + loss4 + neg


_finish = pl.pallas_call(
    _finish_body,
    out_shape=jax.ShapeDtypeStruct((1, 1), jnp.float32),
    out_specs=pl.BlockSpec(memory_space=pltpu.SMEM),
)


def kernel(classEmb, relEmb, nf1, nf2, nf3, nf4, disjoint, nf3_neg):
    parts = _sc_kernel(classEmb, relEmb,
                       nf1.reshape(-1), nf2.reshape(-1), nf3.reshape(-1),
                       nf4.reshape(-1), disjoint.reshape(-1),
                       nf3_neg.reshape(-1))
    return _finish(*parts).reshape(())


# speedup vs baseline: 1.1560x; 1.1560x over previous
import jax
import jax.numpy as jnp
from jax import lax
from jax.experimental import pallas as pl
from jax.experimental.pallas import tpu as pltpu
from jax.experimental.pallas import tpu_sc as plsc

DIM = 128
BATCH = 512
L = 16
NC, NS = 2, 16
NW = NC * NS
RPW = BATCH // NW
NCHUNK = DIM // L


def _sc_body(cE, rE, nf1, nf2, nf3, nf4, dj, neg,
             o1, o2a, o2b, o3, o4, odj, oneg,
             ib1, ib2, ib3, ib4, ibdj, ibng,
             a1, b1, a2, b2, e2b, a3, b3, r3, a4, b4, r4,
             adj, bdj, ang, bng, rng,
             acc_a, acc_b, isem, sems):
    cid = lax.axis_index("c")
    sid = lax.axis_index("s")
    wid = sid * NC + cid
    base = wid * RPW
    iota = lax.iota(jnp.int32, L)

    idx_cps = []
    for src, dst in ((nf1, ib1), (nf2, ib2), (nf3, ib3), (nf4, ib4),
                     (dj, ibdj), (neg, ibng)):
        cp = pltpu.make_async_copy(src.at[pl.ds(base, RPW)], dst, isem)
        cp.start()
        idx_cps.append(cp)
    for cp in idx_cps:
        cp.wait()

    def col(ib, j):
        return plsc.load_gather(ib, [iota, jnp.full((L,), j, jnp.int32)])

    plans = [
        (sems.at[0], ((cE, col(ib1, 0), a1), (cE, col(ib1, 1), b1))),
        (sems.at[1], ((cE, col(ibdj, 0), adj), (cE, col(ibdj, 1), bdj))),
        (sems.at[2], ((cE, col(ib3, 0), a3), (cE, col(ib3, 2), b3),
                      (rE, col(ib3, 1), r3))),
        (sems.at[3], ((cE, col(ibng, 0), ang), (cE, col(ibng, 2), bng),
                      (rE, col(ibng, 1), rng))),
        (sems.at[4], ((cE, col(ib4, 1), a4), (cE, col(ib4, 2), b4),
                      (rE, col(ib4, 0), r4))),
        (sems.at[5], ((cE, col(ib2, 0), a2), (cE, col(ib2, 1), b2),
                      (cE, col(ib2, 2), e2b))),
    ]
    started = []
    for sem, gathers in plans:
        cps = [pltpu.make_async_copy(tab.at[ix], buf, sem)
               for tab, ix, buf in gathers]
        for cp in cps:
            cp.start()
        started.append(cps)

    def wait(k):
        for cp in started[k]:
            cp.wait()

    def cc_loss(out_ref, cbuf, dbuf, rbuf, r_sign, co_sign):
        def row(i, _):
            def chunk(k, acc):
                c1 = cbuf[i, pl.ds(k * L, L)]
                d1 = dbuf[i, pl.ds(k * L, L)]
                co = jnp.abs(cbuf[i, pl.ds(DIM + k * L, L)])
                do = jnp.abs(dbuf[i, pl.ds(DIM + k * L, L)])
                cen = c1 - d1
                if rbuf is not None:
                    r = rbuf[i, pl.ds(k * L, L)]
                    cen = cen + r if r_sign > 0 else cen - r
                euc = jnp.abs(cen)
                if co_sign > 0:
                    t = jnp.maximum(euc + co - do, 0.0)
                else:
                    t = jnp.maximum(euc - co - do, 0.0)
                return acc + t * t
            acc = lax.fori_loop(0, NCHUNK, chunk, jnp.zeros((L,), jnp.float32),
                                unroll=True)
            acc_a[i, :] = acc
            return 0
        lax.fori_loop(0, RPW, row, 0)
        pltpu.sync_copy(acc_a, out_ref.at[pl.ds(base, RPW)])

    def dj_loss(out_ref, cbuf, dbuf):
        def row(i, _):
            def chunk(k, acc):
                c1 = cbuf[i, pl.ds(k * L, L)]
                d1 = dbuf[i, pl.ds(k * L, L)]
                co = jnp.abs(cbuf[i, pl.ds(DIM + k * L, L)])
                do = jnp.abs(dbuf[i, pl.ds(DIM + k * L, L)])
                t = jnp.maximum(co + do - jnp.abs(c1 - d1), 0.0)
                return acc + t * t
            acc = lax.fori_loop(0, NCHUNK, chunk, jnp.zeros((L,), jnp.float32),
                                unroll=True)
            acc_a[i, :] = acc
            return 0
        lax.fori_loop(0, RPW, row, 0)
        pltpu.sync_copy(acc_a, out_ref.at[pl.ds(base, RPW)])

    wait(0)
    cc_loss(o1, a1, b1, None, 0, +1)
    wait(1)
    dj_loss(odj, adj, bdj)
    wait(2)
    cc_loss(o3, a3, b3, r3, +1, +1)
    wait(3)
    cc_loss(oneg, ang, bng, rng, +1, -1)
    wait(4)
    cc_loss(o4, a4, b4, r4, -1, -1)
    wait(5)

    def nf2_row(i, _):
        def chunk(k, carry):
            aa, bb = carry
            c1 = a2[i, pl.ds(k * L, L)]
            d1 = b2[i, pl.ds(k * L, L)]
            e1 = e2b[i, pl.ds(k * L, L)]
            c2 = jnp.abs(a2[i, pl.ds(DIM + k * L, L)])
            d2 = jnp.abs(b2[i, pl.ds(DIM + k * L, L)])
            e2 = jnp.abs(e2b[i, pl.ds(DIM + k * L, L)])
            start = jnp.maximum(c1 - c2, d1 - d2)
            end = jnp.minimum(c1 + c2, d1 + d2)
            diff = start - end
            new_r = jnp.abs(diff) * 0.5
            cen1 = (start + end) * 0.5
            u = jnp.maximum(jnp.abs(cen1 - e1) + new_r - e2, 0.0)
            v = jnp.maximum(diff, 0.0)
            return aa + u * u, bb + v * v
        zero = jnp.zeros((L,), jnp.float32)
        aa, bb = lax.fori_loop(0, NCHUNK, chunk, (zero, zero), unroll=True)
        acc_a[i, :] = aa
        acc_b[i, :] = bb
        return 0
    lax.fori_loop(0, RPW, nf2_row, 0)
    pltpu.sync_copy(acc_a, o2a.at[pl.ds(base, RPW)])
    pltpu.sync_copy(acc_b, o2b.at[pl.ds(base, RPW)])


_part = jax.ShapeDtypeStruct((BATCH, L), jnp.float32)
_cbuf = pltpu.VMEM((RPW, 2 * DIM), jnp.float32)
_rbuf = pltpu.VMEM((RPW, DIM), jnp.float32)

_sc_kernel = pl.kernel(
    _sc_body,
    out_type=(_part,) * 7,
    mesh=plsc.VectorSubcoreMesh(core_axis_name="c", subcore_axis_name="s"),
    compiler_params=pltpu.CompilerParams(needs_layout_passes=False),
    scratch_types=[
        pltpu.VMEM((RPW, 2), jnp.int32),
        pltpu.VMEM((RPW, 3), jnp.int32),
        pltpu.VMEM((RPW, 3), jnp.int32),
        pltpu.VMEM((RPW, 3), jnp.int32),
        pltpu.VMEM((RPW, 2), jnp.int32),
        pltpu.VMEM((RPW, 3), jnp.int32),
        _cbuf, _cbuf,
        _cbuf, _cbuf, _cbuf,
        _cbuf, _cbuf, _rbuf,
        _cbuf, _cbuf, _rbuf,
        _cbuf, _cbuf,
        _cbuf, _cbuf, _rbuf,
        pltpu.VMEM((RPW, L), jnp.float32),
        pltpu.VMEM((RPW, L), jnp.float32),
        pltpu.SemaphoreType.DMA,
        pltpu.SemaphoreType.DMA((6,)),
    ],
)


def _finish_body(o1, o2a, o2b, o3, o4, odj, oneg, out):
    inv_b = 1.0 / BATCH
    loss1 = jnp.sum(o1[...]) * inv_b
    loss3 = jnp.sum(o3[...]) * inv_b
    loss4 = jnp.sum(o4[...]) * inv_b
    dj = jnp.sum(odj[...]) * inv_b
    a2 = jnp.sum(o2a[...], axis=1, keepdims=True)
    b2 = jnp.sum(o2b[...], axis=1, keepdims=True)
    mean_a = jnp.sum(jnp.sqrt(a2)) * inv_b
    mean_b = jnp.sum(jnp.sqrt(b2)) * inv_b
    loss2 = (jnp.sum(a2) + jnp.sum(b2)) * inv_b + 2.0 * mean_a * mean_b
    n2 = jnp.sum(oneg[...], axis=1, keepdims=True)
    dn = jnp.sqrt(n2)
    neg = jnp.sum((dn - 2.0) ** 2) * inv_b
    out[0, 0] = loss1 + loss2 + dj + loss3 + loss4 + neg


_finish = pl.pallas_call(
    _finish_body,
    out_shape=jax.ShapeDtypeStruct((1, 1), jnp.float32),
    out_specs=pl.BlockSpec(memory_space=pltpu.SMEM),
)


def kernel(classEmb, relEmb, nf1, nf2, nf3, nf4, disjoint, nf3_neg):
    parts = _sc_kernel(classEmb, relEmb, nf1, nf2, nf3, nf4,
                       disjoint, nf3_neg)
    return _finish(*parts).reshape(())


# speedup vs baseline: 1.5227x; 1.3173x over previous
import jax
import jax.numpy as jnp
from jax import lax
from jax.experimental import pallas as pl
from jax.experimental.pallas import tpu as pltpu
from jax.experimental.pallas import tpu_sc as plsc

DIM = 128
BATCH = 512
L = 16
NC, NS = 2, 16
NW = NC * NS
RPW = BATCH // NW
NCHUNK = DIM // L


def _sc_body(cE, rE, idx_all,
             o1, o2a, o2b, o3, o4, odj, oneg,
             ib,
             a1, b1, a2, b2, e2b, a3, b3, r3, a4, b4, r4,
             adj, bdj, ang, bng, rng,
             p1, pdj, p3, png, p4, p2a, p2b,
             isem, osem, sems):
    cid = lax.axis_index("c")
    sid = lax.axis_index("s")
    wid = sid * NC + cid
    base = wid * RPW
    iota = lax.iota(jnp.int32, L)

    _icp = pltpu.make_async_copy(idx_all.at[pl.ds(base, RPW)], ib, isem)
    _icp.start()
    _icp.wait()

    def col(j):
        return plsc.load_gather(ib, [iota, jnp.full((L,), j, jnp.int32)])

    plans = [
        (sems.at[0], ((cE, col(0), a1), (cE, col(1), b1))),
        (sems.at[1], ((cE, col(11), adj), (cE, col(12), bdj))),
        (sems.at[2], ((cE, col(5), a3), (cE, col(7), b3), (rE, col(6), r3))),
        (sems.at[3], ((cE, col(13), ang), (cE, col(15), bng),
                      (rE, col(14), rng))),
        (sems.at[4], ((cE, col(9), a4), (cE, col(10), b4), (rE, col(8), r4))),
        (sems.at[5], ((cE, col(2), a2), (cE, col(3), b2), (cE, col(4), e2b))),
    ]
    started = []
    for sem, gathers in plans:
        cps = [pltpu.make_async_copy(tab.at[ix], buf, sem)
               for tab, ix, buf in gathers]
        for cp in cps:
            cp.start()
        started.append(cps)

    def wait(k):
        for cp in started[k]:
            cp.wait()

    out_cps = []

    def emit(src, dst):
        cp = pltpu.make_async_copy(src, dst.at[pl.ds(base, RPW)], osem)
        cp.start()
        out_cps.append(cp)

    def cc_loss(out_buf, cbuf, dbuf, rbuf, r_sign, co_sign):
        def row(i, _):
            def chunk(k, carry):
                accs = []
                for h, acc in enumerate(carry):
                    kk = 2 * k + h
                    c1 = cbuf[i, pl.ds(kk * L, L)]
                    d1 = dbuf[i, pl.ds(kk * L, L)]
                    co = jnp.abs(cbuf[i, pl.ds(DIM + kk * L, L)])
                    do = jnp.abs(dbuf[i, pl.ds(DIM + kk * L, L)])
                    cen = c1 - d1
                    if rbuf is not None:
                        r = rbuf[i, pl.ds(kk * L, L)]
                        cen = cen + r if r_sign > 0 else cen - r
                    euc = jnp.abs(cen)
                    if co_sign > 0:
                        t = jnp.maximum(euc + co - do, 0.0)
                    else:
                        t = jnp.maximum(euc - co - do, 0.0)
                    accs.append(acc + t * t)
                return tuple(accs)
            zero = jnp.zeros((L,), jnp.float32)
            acc0, acc1 = lax.fori_loop(0, NCHUNK // 2, chunk, (zero, zero),
                                       unroll=True)
            out_buf[i, :] = acc0 + acc1
            return 0
        lax.fori_loop(0, RPW, row, 0)

    wait(0)
    cc_loss(p1, a1, b1, None, 0, +1)
    emit(p1, o1)
    wait(1)

    def dj_row(i, _):
        def chunk(k, acc):
            c1 = adj[i, pl.ds(k * L, L)]
            d1 = bdj[i, pl.ds(k * L, L)]
            co = jnp.abs(adj[i, pl.ds(DIM + k * L, L)])
            do = jnp.abs(bdj[i, pl.ds(DIM + k * L, L)])
            t = jnp.maximum(co + do - jnp.abs(c1 - d1), 0.0)
            return acc + t * t
        acc = lax.fori_loop(0, NCHUNK, chunk, jnp.zeros((L,), jnp.float32),
                            unroll=True)
        pdj[i, :] = acc
        return 0
    lax.fori_loop(0, RPW, dj_row, 0)
    emit(pdj, odj)

    wait(2)
    cc_loss(p3, a3, b3, r3, +1, +1)
    emit(p3, o3)
    wait(3)
    cc_loss(png, ang, bng, rng, +1, -1)
    emit(png, oneg)
    wait(4)
    cc_loss(p4, a4, b4, r4, -1, -1)
    emit(p4, o4)
    wait(5)

    def nf2_row(i, _):
        def chunk(k, carry):
            aa, bb = carry
            c1 = a2[i, pl.ds(k * L, L)]
            d1 = b2[i, pl.ds(k * L, L)]
            e1 = e2b[i, pl.ds(k * L, L)]
            c2 = jnp.abs(a2[i, pl.ds(DIM + k * L, L)])
            d2 = jnp.abs(b2[i, pl.ds(DIM + k * L, L)])
            e2 = jnp.abs(e2b[i, pl.ds(DIM + k * L, L)])
            start = jnp.maximum(c1 - c2, d1 - d2)
            end = jnp.minimum(c1 + c2, d1 + d2)
            diff = start - end
            new_r = jnp.abs(diff) * 0.5
            cen1 = (start + end) * 0.5
            u = jnp.maximum(jnp.abs(cen1 - e1) + new_r - e2, 0.0)
            v = jnp.maximum(diff, 0.0)
            return aa + u * u, bb + v * v
        zero = jnp.zeros((L,), jnp.float32)
        aa, bb = lax.fori_loop(0, NCHUNK, chunk, (zero, zero), unroll=True)
        p2a[i, :] = aa
        p2b[i, :] = bb
        return 0
    lax.fori_loop(0, RPW, nf2_row, 0)
    emit(p2a, o2a)
    emit(p2b, o2b)

    for cp in out_cps:
        cp.wait()


_part = jax.ShapeDtypeStruct((BATCH, L), jnp.float32)
_cbuf = pltpu.VMEM((RPW, 2 * DIM), jnp.float32)
_rbuf = pltpu.VMEM((RPW, DIM), jnp.float32)
_pbuf = pltpu.VMEM((RPW, L), jnp.float32)

_sc_kernel = pl.kernel(
    _sc_body,
    out_type=(_part,) * 7,
    mesh=plsc.VectorSubcoreMesh(core_axis_name="c", subcore_axis_name="s"),
    compiler_params=pltpu.CompilerParams(needs_layout_passes=False),
    scratch_types=[
        pltpu.VMEM((RPW, 16), jnp.int32),
        _cbuf, _cbuf,
        _cbuf, _cbuf, _cbuf,
        _cbuf, _cbuf, _rbuf,
        _cbuf, _cbuf, _rbuf,
        _cbuf, _cbuf,
        _cbuf, _cbuf, _rbuf,
        _pbuf, _pbuf, _pbuf, _pbuf, _pbuf,
        _pbuf, _pbuf,
        pltpu.SemaphoreType.DMA,
        pltpu.SemaphoreType.DMA,
        pltpu.SemaphoreType.DMA((6,)),
    ],
)


def _finish_body(o1, o2a, o2b, o3, o4, odj, oneg, out):
    inv_b = 1.0 / BATCH
    loss1 = jnp.sum(o1[...]) * inv_b
    loss3 = jnp.sum(o3[...]) * inv_b
    loss4 = jnp.sum(o4[...]) * inv_b
    dj = jnp.sum(odj[...]) * inv_b
    a2 = jnp.sum(o2a[...], axis=1, keepdims=True)
    b2 = jnp.sum(o2b[...], axis=1, keepdims=True)
    mean_a = jnp.sum(jnp.sqrt(a2)) * inv_b
    mean_b = jnp.sum(jnp.sqrt(b2)) * inv_b
    loss2 = (jnp.sum(a2) + jnp.sum(b2)) * inv_b + 2.0 * mean_a * mean_b
    n2 = jnp.sum(oneg[...], axis=1, keepdims=True)
    dn = jnp.sqrt(n2)
    neg = jnp.sum((dn - 2.0) ** 2) * inv_b
    out[0, 0] = loss1 + loss2 + dj + loss3 + loss4 + neg


_finish = pl.pallas_call(
    _finish_body,
    out_shape=jax.ShapeDtypeStruct((1, 1), jnp.float32),
    out_specs=pl.BlockSpec(memory_space=pltpu.SMEM),
)


def kernel(classEmb, relEmb, nf1, nf2, nf3, nf4, disjoint, nf3_neg):
    idx_all = jnp.concatenate(
        [nf1[:BATCH], nf2[:BATCH], nf3[:BATCH], nf4[:BATCH],
         disjoint[:BATCH], nf3_neg[:BATCH]], axis=1)
    parts = _sc_kernel(classEmb, relEmb, idx_all)
    return _finish(*parts).reshape(())
